# baseline (speedup 1.0000x reference)
import math

import jax
import jax.numpy as jnp
from jax.experimental import pallas as pl
from jax.experimental.pallas import tpu as pltpu

_N_VOCAB = 1000000
_N_DIM = 64
_SIGMA = 1.0

_BLOCK_COLS = 8192
_NUM_BLOCKS = -(-_N_VOCAB // _BLOCK_COLS)
_TAIL_COLS = _N_VOCAB - (_NUM_BLOCKS - 1) * _BLOCK_COLS

_N_ELEMS = 2 * _N_VOCAB * _N_DIM
_CONST = _N_ELEMS * (-math.log(_SIGMA) - 0.5 * math.log(2.0 * math.pi))


def _accumulate(acc, w, c):
    for k in range(_BLOCK_COLS // 128):
        ws = w[:, k * 128 : (k + 1) * 128]
        cs = c[:, k * 128 : (k + 1) * 128]
        acc = acc + ws * ws + cs * cs
    return acc


def _prior_body(w_ref, c_ref, o_ref, acc_ref):
    i = pl.program_id(0)

    @pl.when(i == 0)
    def _init():
        acc_ref[...] = jnp.zeros((_N_DIM, 128), jnp.float32)

    @pl.when(i < _NUM_BLOCKS - 1)
    def _full_block():
        acc_ref[...] = _accumulate(acc_ref[...], w_ref[...], c_ref[...])

    @pl.when(i == _NUM_BLOCKS - 1)
    def _ragged_block_and_finalize():
        lane = jax.lax.broadcasted_iota(jnp.int32, (_N_DIM, 128), 1)
        acc = acc_ref[...]
        for k in range(-(-_TAIL_COLS // 128)):
            sl = slice(k * 128, (k + 1) * 128)
            ws = w_ref[:, sl]
            cs = c_ref[:, sl]
            valid = _TAIL_COLS - k * 128
            if valid < 128:
                m = lane < valid
                ws = jnp.where(m, ws, 0.0)
                cs = jnp.where(m, cs, 0.0)
            acc = acc + ws * ws + cs * cs
        o_ref[0, 0] = -0.5 * jnp.sum(acc) + _CONST


def kernel(target_ixs, context_ixs, negative_sample_ixs, word_weight, context_weight):
    del target_ixs, context_ixs, negative_sample_ixs
    w = word_weight.T
    c = context_weight.T

    out = pl.pallas_call(
        _prior_body,
        grid=(_NUM_BLOCKS,),
        in_specs=[
            pl.BlockSpec((_N_DIM, _BLOCK_COLS), lambda i: (0, i)),
            pl.BlockSpec((_N_DIM, _BLOCK_COLS), lambda i: (0, i)),
        ],
        out_specs=pl.BlockSpec(
            (1, 1), lambda i: (0, 0), memory_space=pltpu.MemorySpace.SMEM
        ),
        out_shape=jax.ShapeDtypeStruct((1, 1), jnp.float32),
        scratch_shapes=[pltpu.VMEM((_N_DIM, 128), jnp.float32)],
    )(w, c)
    return out[0, 0]


# speedup vs baseline: 1.1080x; 1.1080x over previous
import math

import jax
import jax.numpy as jnp
from jax import lax
from jax.experimental import pallas as pl
from jax.experimental.pallas import tpu as pltpu

_N_VOCAB = 1000000
_N_DIM = 64
_SIGMA = 1.0

_SMALL = 2048
_N_SMALL = 8
_CHUNK = 16384
_NBUF = 5
_BIG0 = _N_SMALL * _SMALL
_N_BIG = (_N_VOCAB - _BIG0) // _CHUNK
_TAIL0 = _BIG0 + _N_BIG * _CHUNK
_TAIL = _N_VOCAB - _TAIL0

_N_ELEMS = 2 * _N_VOCAB * _N_DIM
_CONST = _N_ELEMS * (-math.log(_SIGMA) - 0.5 * math.log(2.0 * math.pi))


def _copy(src_hbm, col, width, dst, sem):
    return pltpu.make_async_copy(src_hbm.at[:, pl.ds(col, width)], dst, sem)


def _accum_slices(acc, w, c, n_slices):
    for k in range(n_slices):
        ws = w[:, k * 128 : (k + 1) * 128]
        cs = c[:, k * 128 : (k + 1) * 128]
        acc = acc + ws * ws + cs * cs
    return acc


def _prior_body(w_hbm, c_hbm, o_ref, swbuf, scbuf, wbuf, cbuf, twbuf, tcbuf,
                ssemw, ssemc, bsemw, bsemc, tsemw, tsemc):
    for k in range(_N_SMALL):
        _copy(w_hbm, k * _SMALL, _SMALL, swbuf.at[k], ssemw.at[k]).start()
        _copy(c_hbm, k * _SMALL, _SMALL, scbuf.at[k], ssemc.at[k]).start()
    for b in range(_NBUF):
        _copy(w_hbm, _BIG0 + b * _CHUNK, _CHUNK, wbuf.at[b], bsemw.at[b]).start()
        _copy(c_hbm, _BIG0 + b * _CHUNK, _CHUNK, cbuf.at[b], bsemc.at[b]).start()
    _copy(w_hbm, _TAIL0, _TAIL, twbuf, tsemw).start()
    _copy(c_hbm, _TAIL0, _TAIL, tcbuf, tsemc).start()

    acc = jnp.zeros((_N_DIM, 128), jnp.float32)

    for k in range(_N_SMALL):
        _copy(w_hbm, k * _SMALL, _SMALL, swbuf.at[k], ssemw.at[k]).wait()
        _copy(c_hbm, k * _SMALL, _SMALL, scbuf.at[k], ssemc.at[k]).wait()
        acc = _accum_slices(acc, swbuf[k], scbuf[k], _SMALL // 128)

    def ring_step(g, acc):
        for b in range(_NBUF):
            col = _BIG0 + (g * _NBUF + b) * _CHUNK
            _copy(w_hbm, col, _CHUNK, wbuf.at[b], bsemw.at[b]).wait()
            _copy(c_hbm, col, _CHUNK, cbuf.at[b], bsemc.at[b]).wait()
            acc = _accum_slices(acc, wbuf[b], cbuf[b], _CHUNK // 128)
            nxt = col + _NBUF * _CHUNK

            @pl.when(g * _NBUF + b + _NBUF < _N_BIG)
            def _prefetch():
                _copy(w_hbm, nxt, _CHUNK, wbuf.at[b], bsemw.at[b]).start()
                _copy(c_hbm, nxt, _CHUNK, cbuf.at[b], bsemc.at[b]).start()
        return acc

    acc = lax.fori_loop(0, _N_BIG // _NBUF, ring_step, acc)

    _copy(w_hbm, _TAIL0, _TAIL, twbuf, tsemw).wait()
    _copy(c_hbm, _TAIL0, _TAIL, tcbuf, tsemc).wait()
    acc = _accum_slices(acc, twbuf[...], tcbuf[...], _TAIL // 128)
    base = (_TAIL // 128) * 128
    ws = twbuf[:, base:_TAIL]
    cs = tcbuf[:, base:_TAIL]
    tail_sum = jnp.sum(ws * ws) + jnp.sum(cs * cs)

    o_ref[0, 0] = -0.5 * (jnp.sum(acc) + tail_sum) + _CONST


def kernel(target_ixs, context_ixs, negative_sample_ixs, word_weight, context_weight):
    del target_ixs, context_ixs, negative_sample_ixs
    w = word_weight.T
    c = context_weight.T

    out = pl.pallas_call(
        _prior_body,
        in_specs=[
            pl.BlockSpec(memory_space=pltpu.MemorySpace.HBM),
            pl.BlockSpec(memory_space=pltpu.MemorySpace.HBM),
        ],
        out_specs=pl.BlockSpec(memory_space=pltpu.MemorySpace.SMEM),
        out_shape=jax.ShapeDtypeStruct((1, 1), jnp.float32),
        scratch_shapes=[
            pltpu.VMEM((_N_SMALL, _N_DIM, _SMALL), jnp.float32),
            pltpu.VMEM((_N_SMALL, _N_DIM, _SMALL), jnp.float32),
            pltpu.VMEM((_NBUF, _N_DIM, _CHUNK), jnp.float32),
            pltpu.VMEM((_NBUF, _N_DIM, _CHUNK), jnp.float32),
            pltpu.VMEM((_N_DIM, _TAIL), jnp.float32),
            pltpu.VMEM((_N_DIM, _TAIL), jnp.float32),
            pltpu.SemaphoreType.DMA((_N_SMALL,)),
            pltpu.SemaphoreType.DMA((_N_SMALL,)),
            pltpu.SemaphoreType.DMA((_NBUF,)),
            pltpu.SemaphoreType.DMA((_NBUF,)),
            pltpu.SemaphoreType.DMA,
            pltpu.SemaphoreType.DMA,
        ],
    )(w, c)
    return out[0, 0]


# speedup vs baseline: 1.1114x; 1.0031x over previous
import math

import jax
import jax.numpy as jnp
from jax.experimental import pallas as pl
from jax.experimental.pallas import tpu as pltpu

_N_VOCAB = 1000000
_N_DIM = 64
_SIGMA = 1.0

_BLOCK_COLS = 32768
_NUM_BLOCKS = -(-_N_VOCAB // _BLOCK_COLS)
_TAIL_COLS = _N_VOCAB - (_NUM_BLOCKS - 1) * _BLOCK_COLS

_N_ELEMS = 2 * _N_VOCAB * _N_DIM
_CONST = _N_ELEMS * (-math.log(_SIGMA) - 0.5 * math.log(2.0 * math.pi))


def _accumulate(acc, w, c):
    for k in range(_BLOCK_COLS // 128):
        ws = w[:, k * 128 : (k + 1) * 128]
        cs = c[:, k * 128 : (k + 1) * 128]
        acc = acc + ws * ws + cs * cs
    return acc


def _prior_body(w_ref, c_ref, o_ref, acc_ref):
    i = pl.program_id(0)

    @pl.when(i == 0)
    def _init():
        acc_ref[...] = jnp.zeros((_N_DIM, 128), jnp.float32)

    @pl.when(i < _NUM_BLOCKS - 1)
    def _full_block():
        acc_ref[...] = _accumulate(acc_ref[...], w_ref[...], c_ref[...])

    @pl.when(i == _NUM_BLOCKS - 1)
    def _ragged_block_and_finalize():
        lane = jax.lax.broadcasted_iota(jnp.int32, (_N_DIM, 128), 1)
        acc = acc_ref[...]
        for k in range(-(-_TAIL_COLS // 128)):
            sl = slice(k * 128, (k + 1) * 128)
            ws = w_ref[:, sl]
            cs = c_ref[:, sl]
            valid = _TAIL_COLS - k * 128
            if valid < 128:
                m = lane < valid
                ws = jnp.where(m, ws, 0.0)
                cs = jnp.where(m, cs, 0.0)
            acc = acc + ws * ws + cs * cs
        o_ref[0, 0] = -0.5 * jnp.sum(acc) + _CONST


def kernel(target_ixs, context_ixs, negative_sample_ixs, word_weight, context_weight):
    del target_ixs, context_ixs, negative_sample_ixs
    w = word_weight.T
    c = context_weight.T

    out = pl.pallas_call(
        _prior_body,
        grid=(_NUM_BLOCKS,),
        in_specs=[
            pl.BlockSpec((_N_DIM, _BLOCK_COLS), lambda i: (0, i)),
            pl.BlockSpec((_N_DIM, _BLOCK_COLS), lambda i: (0, i)),
        ],
        out_specs=pl.BlockSpec(
            (1, 1), lambda i: (0, 0), memory_space=pltpu.MemorySpace.SMEM
        ),
        out_shape=jax.ShapeDtypeStruct((1, 1), jnp.float32),
        scratch_shapes=[pltpu.VMEM((_N_DIM, 128), jnp.float32)],
    )(w, c)
    return out[0, 0]


# speedup vs baseline: 1.1151x; 1.0033x over previous
import math

import jax
import jax.numpy as jnp
from jax.experimental import pallas as pl
from jax.experimental.pallas import tpu as pltpu

_N_VOCAB = 1000000
_N_DIM = 64
_SIGMA = 1.0

_BLOCK_COLS = 49152
_NUM_BLOCKS = -(-_N_VOCAB // _BLOCK_COLS)
_TAIL_COLS = _N_VOCAB - (_NUM_BLOCKS - 1) * _BLOCK_COLS

_N_ELEMS = 2 * _N_VOCAB * _N_DIM
_CONST = _N_ELEMS * (-math.log(_SIGMA) - 0.5 * math.log(2.0 * math.pi))


def _accumulate(acc, w, c):
    for k in range(_BLOCK_COLS // 128):
        ws = w[:, k * 128 : (k + 1) * 128]
        cs = c[:, k * 128 : (k + 1) * 128]
        acc = acc + ws * ws + cs * cs
    return acc


def _prior_body(w_ref, c_ref, o_ref, acc_ref):
    i = pl.program_id(0)

    @pl.when(i == 0)
    def _init():
        acc_ref[...] = jnp.zeros((_N_DIM, 128), jnp.float32)

    @pl.when(i < _NUM_BLOCKS - 1)
    def _full_block():
        acc_ref[...] = _accumulate(acc_ref[...], w_ref[...], c_ref[...])

    @pl.when(i == _NUM_BLOCKS - 1)
    def _ragged_block_and_finalize():
        lane = jax.lax.broadcasted_iota(jnp.int32, (_N_DIM, 128), 1)
        acc = acc_ref[...]
        for k in range(-(-_TAIL_COLS // 128)):
            sl = slice(k * 128, (k + 1) * 128)
            ws = w_ref[:, sl]
            cs = c_ref[:, sl]
            valid = _TAIL_COLS - k * 128
            if valid < 128:
                m = lane < valid
                ws = jnp.where(m, ws, 0.0)
                cs = jnp.where(m, cs, 0.0)
            acc = acc + ws * ws + cs * cs
        o_ref[0, 0] = -0.5 * jnp.sum(acc) + _CONST


def kernel(target_ixs, context_ixs, negative_sample_ixs, word_weight, context_weight):
    del target_ixs, context_ixs, negative_sample_ixs
    w = word_weight.T
    c = context_weight.T

    out = pl.pallas_call(
        _prior_body,
        grid=(_NUM_BLOCKS,),
        in_specs=[
            pl.BlockSpec((_N_DIM, _BLOCK_COLS), lambda i: (0, i)),
            pl.BlockSpec((_N_DIM, _BLOCK_COLS), lambda i: (0, i)),
        ],
        out_specs=pl.BlockSpec(
            (1, 1), lambda i: (0, 0), memory_space=pltpu.MemorySpace.SMEM
        ),
        out_shape=jax.ShapeDtypeStruct((1, 1), jnp.float32),
        scratch_shapes=[pltpu.VMEM((_N_DIM, 128), jnp.float32)],
    )(w, c)
    return out[0, 0]
